# Initial kernel scaffold; baseline (speedup 1.0000x reference)
import jax
import jax.numpy as jnp
from jax.experimental import pallas as pl


def kernel(output, y):
    raise NotImplementedError("write your pallas kernel here")


# speedup vs baseline: 3.0729x; 3.0729x over previous
import functools

import jax
import jax.numpy as jnp
from jax.experimental import pallas as pl


def _hinge_body(out_ref, y_ref, loss_ref, *, n_classes):
    out = out_ref[...]
    y = y_ref[...]
    cols = jax.lax.broadcasted_iota(jnp.int32, out.shape, 1)
    mask = cols == y[:, None]
    out_y = jnp.sum(jnp.where(mask, out, 0.0), axis=1, keepdims=True)
    s = jnp.sum(jnp.maximum(out - out_y + 1.0, 0.0), axis=1)
    loss_ref[...] = (s - 1.0) * (1.0 / n_classes)


def kernel(output, y):
    b, c = output.shape
    y = y.astype(jnp.int32)
    br = 512
    grid = (b // br,)
    body = functools.partial(_hinge_body, n_classes=c)
    return pl.pallas_call(
        body,
        grid=grid,
        in_specs=[
            pl.BlockSpec((br, c), lambda i: (i, 0)),
            pl.BlockSpec((br,), lambda i: (i,)),
        ],
        out_specs=pl.BlockSpec((br,), lambda i: (i,)),
        out_shape=jax.ShapeDtypeStruct((b,), jnp.float32),
    )(output, y)


# speedup vs baseline: 3.3733x; 1.0978x over previous
import functools

import jax
import jax.numpy as jnp
from jax.experimental import pallas as pl


def _hinge_body(out_ref, y_ref, loss_ref, *, n_classes):
    out = out_ref[...]
    y = y_ref[...]
    cols = jax.lax.broadcasted_iota(jnp.int32, out.shape, 1)
    mask = cols == y[:, None]
    out_y = jnp.sum(jnp.where(mask, out, 0.0), axis=1, keepdims=True)
    s = jnp.sum(jnp.maximum(out - out_y + 1.0, 0.0), axis=1)
    loss_ref[...] = (s - 1.0) * (1.0 / n_classes)


def kernel(output, y):
    b, c = output.shape
    y = y.astype(jnp.int32)
    br = 1024
    grid = (b // br,)
    body = functools.partial(_hinge_body, n_classes=c)
    return pl.pallas_call(
        body,
        grid=grid,
        in_specs=[
            pl.BlockSpec((br, c), lambda i: (i, 0)),
            pl.BlockSpec((br,), lambda i: (i,)),
        ],
        out_specs=pl.BlockSpec((br,), lambda i: (i,)),
        out_shape=jax.ShapeDtypeStruct((b,), jnp.float32),
    )(output, y)


# speedup vs baseline: 10.1943x; 3.0220x over previous
import functools

import jax
import jax.numpy as jnp
from jax.experimental import pallas as pl


def _hinge_body(xt_ref, y_ref, loss_ref, *, n_classes):
    xt = xt_ref[...]
    y = y_ref[...]
    ci = jax.lax.broadcasted_iota(jnp.int32, xt.shape, 0)
    mask = ci == y[None, :]
    out_y = jnp.sum(jnp.where(mask, xt, 0.0), axis=0)
    s = jnp.sum(jnp.maximum(xt - out_y[None, :] + 1.0, 0.0), axis=0)
    loss_ref[...] = (s - 1.0) * (1.0 / n_classes)


def kernel(output, y):
    b, c = output.shape
    y = y.astype(jnp.int32)
    xt = output.T
    bl = 2048
    grid = (b // bl,)
    body = functools.partial(_hinge_body, n_classes=c)
    return pl.pallas_call(
        body,
        grid=grid,
        in_specs=[
            pl.BlockSpec((c, bl), lambda i: (0, i)),
            pl.BlockSpec((bl,), lambda i: (i,)),
        ],
        out_specs=pl.BlockSpec((bl,), lambda i: (i,)),
        out_shape=jax.ShapeDtypeStruct((b,), jnp.float32),
    )(xt, y)


# speedup vs baseline: 12.9290x; 1.2683x over previous
import functools

import jax
import jax.numpy as jnp
from jax.experimental import pallas as pl


def _hinge_body(xt_ref, y_ref, loss_ref, *, n_classes, bl):
    ch = 8
    y = y_ref[...]
    sub = jax.lax.broadcasted_iota(jnp.int32, (ch, bl), 0)
    d = y[None, :] - sub

    acc_y = jnp.zeros((ch, bl), jnp.float32)
    for k in range(n_classes // ch):
        xk = xt_ref[k * ch:(k + 1) * ch, :]
        acc_y = acc_y + jnp.where(d == ch * k, xk, 0.0)
    t = jnp.sum(acc_y, axis=0) - 1.0

    acc_s = jnp.zeros((ch, bl), jnp.float32)
    for k in range(n_classes // ch):
        xk = xt_ref[k * ch:(k + 1) * ch, :]
        acc_s = acc_s + jnp.maximum(xk - t[None, :], 0.0)
    s = jnp.sum(acc_s, axis=0)

    loss_ref[...] = (s - 1.0) * (1.0 / n_classes)


def kernel(output, y):
    b, c = output.shape
    y = y.astype(jnp.int32)
    xt = output.T
    bl = 2048
    grid = (b // bl,)
    body = functools.partial(_hinge_body, n_classes=c, bl=bl)
    return pl.pallas_call(
        body,
        grid=grid,
        in_specs=[
            pl.BlockSpec((c, bl), lambda i: (0, i)),
            pl.BlockSpec((bl,), lambda i: (i,)),
        ],
        out_specs=pl.BlockSpec((bl,), lambda i: (i,)),
        out_shape=jax.ShapeDtypeStruct((b,), jnp.float32),
    )(xt, y)
